# Initial kernel scaffold; baseline (speedup 1.0000x reference)
import jax
import jax.numpy as jnp
from jax.experimental import pallas as pl


def kernel(distance_matrix, table):
    raise NotImplementedError("write your pallas kernel here")


# speedup vs baseline: 2.7667x; 2.7667x over previous
import functools

import jax
import jax.numpy as jnp
from jax import lax
from jax.experimental import pallas as pl
from jax.experimental.pallas import tpu as pltpu
from jax.experimental.pallas import tpu_sc as plsc

B, N, EMB = 1024, 32, 64
NUM_BUCKETS = 201
TOTAL = B * N * N

NC, NS = 2, 16
NW = NC * NS
PER_W = TOTAL // NW
CHUNK = 1024
NCHUNK = PER_W // CHUNK
GSIZE = 128
NG = CHUNK // GSIZE


def _body(dist_hbm, table_hbm, out_hbm, dist_v, idx_v, rows_v, sem):
    wid = lax.axis_index("s") * NC + lax.axis_index("c")
    base = wid * PER_W

    def chunk_body(g, carry):
        off = base + g * CHUNK
        pltpu.sync_copy(dist_hbm.at[pl.ds(off, CHUNK)], dist_v)

        def idx_body(i, carry2):
            v = dist_v[pl.ds(i * 16, 16)]
            iv = jnp.clip(v, 0.0, float(NUM_BUCKETS - 1)).astype(jnp.int32)
            idx_v[pl.ds(i * 16, 16)] = iv
            return carry2

        lax.fori_loop(0, CHUNK // 16, idx_body, 0)

        copies = [
            pltpu.async_copy(
                table_hbm.at[idx_v.at[pl.ds(j * GSIZE, GSIZE)]],
                rows_v.at[pl.ds(j * GSIZE, GSIZE)],
                sem,
            )
            for j in range(NG)
        ]
        for c in copies:
            c.wait()
        pltpu.sync_copy(rows_v, out_hbm.at[pl.ds(off, CHUNK)])
        return carry

    lax.fori_loop(0, NCHUNK, chunk_body, 0)


def kernel(distance_matrix, table):
    dist_flat = distance_matrix.reshape(TOTAL)
    mesh = plsc.VectorSubcoreMesh(core_axis_name="c", subcore_axis_name="s")
    k = functools.partial(
        pl.kernel,
        out_type=jax.ShapeDtypeStruct((TOTAL, EMB), jnp.float32),
        mesh=mesh,
        scratch_types=[
            pltpu.VMEM((CHUNK,), jnp.float32),
            pltpu.VMEM((CHUNK,), jnp.int32),
            pltpu.VMEM((CHUNK, EMB), jnp.float32),
            pltpu.SemaphoreType.DMA,
        ],
        compiler_params=pltpu.CompilerParams(use_tc_tiling_on_sc=False),
    )(_body)
    out = k(dist_flat, table)
    return out.reshape(B, N, N, EMB)


# speedup vs baseline: 4.4800x; 1.6193x over previous
import functools

import jax
import jax.numpy as jnp
from jax import lax
from jax.experimental import pallas as pl
from jax.experimental.pallas import tpu as pltpu
from jax.experimental.pallas import tpu_sc as plsc

B, N, EMB = 1024, 32, 64
NUM_BUCKETS = 201
TOTAL = B * N * N

NC, NS = 2, 16
NW = NC * NS
PER_W = TOTAL // NW
CHUNK = 1024
NCHUNK = PER_W // CHUNK
GSIZE = 128
NG = CHUNK // GSIZE


def _body(dist_hbm, table_hbm, out_hbm, dist_v, idx_v, rows_v, table_v, sem):
    wid = lax.axis_index("s") * NC + lax.axis_index("c")
    base = wid * PER_W

    @pl.when(lax.axis_index("s") == 0)
    def _stage_table():
        pltpu.sync_copy(table_hbm, table_v)

    plsc.subcore_barrier()

    def chunk_body(g, carry):
        off = base + g * CHUNK
        pltpu.sync_copy(dist_hbm.at[pl.ds(off, CHUNK)], dist_v)

        def idx_body(i, carry2):
            v = dist_v[pl.ds(i * 16, 16)]
            iv = jnp.clip(v, 0.0, float(NUM_BUCKETS - 1)).astype(jnp.int32)
            idx_v[pl.ds(i * 16, 16)] = iv
            return carry2

        lax.fori_loop(0, CHUNK // 16, idx_body, 0)

        copies = [
            pltpu.async_copy(
                table_v.at[idx_v.at[pl.ds(j * GSIZE, GSIZE)]],
                rows_v.at[pl.ds(j * GSIZE, GSIZE)],
                sem,
            )
            for j in range(NG)
        ]
        for c in copies:
            c.wait()
        pltpu.sync_copy(rows_v, out_hbm.at[pl.ds(off, CHUNK)])
        return carry

    lax.fori_loop(0, NCHUNK, chunk_body, 0)


def kernel(distance_matrix, table):
    dist_flat = distance_matrix.reshape(TOTAL)
    mesh = plsc.VectorSubcoreMesh(core_axis_name="c", subcore_axis_name="s")
    k = functools.partial(
        pl.kernel,
        out_type=jax.ShapeDtypeStruct((TOTAL, EMB), jnp.float32),
        mesh=mesh,
        scratch_types=[
            pltpu.VMEM((CHUNK,), jnp.float32),
            pltpu.VMEM((CHUNK,), jnp.int32),
            pltpu.VMEM((CHUNK, EMB), jnp.float32),
            pltpu.VMEM_SHARED((NUM_BUCKETS, EMB), jnp.float32),
            pltpu.SemaphoreType.DMA,
        ],
        compiler_params=pltpu.CompilerParams(use_tc_tiling_on_sc=False),
    )(_body)
    out = k(dist_flat, table)
    return out.reshape(B, N, N, EMB)


# speedup vs baseline: 4.9519x; 1.1053x over previous
import functools

import jax
import jax.numpy as jnp
from jax import lax
from jax.experimental import pallas as pl
from jax.experimental.pallas import tpu as pltpu
from jax.experimental.pallas import tpu_sc as plsc

B, N, EMB = 1024, 32, 64
NUM_BUCKETS = 201
TOTAL = B * N * N

NC, NS = 2, 16
NW = NC * NS
PER_W = TOTAL // NW
CHUNK = 512
NCHUNK = PER_W // CHUNK
GSIZE = 128
NG = CHUNK // GSIZE


def _body(dist_hbm, table_hbm, out_hbm,
          dist0, dist1, idx0, idx1, rows0, rows1, table_s,
          dist_sem, gather_sem, store_sem):
    base = (lax.axis_index("s") * NC + lax.axis_index("c")) * PER_W
    dist_v = (dist0, dist1)
    idx_v = (idx0, idx1)
    rows_v = (rows0, rows1)

    @pl.when(lax.axis_index("s") == 0)
    def _stage_table():
        pltpu.sync_copy(table_hbm, table_s)

    plsc.subcore_barrier()

    def compute_idx(b):
        def idx_body(i, carry):
            v = dist_v[b][pl.ds(i * 16, 16)]
            idx_v[b][pl.ds(i * 16, 16)] = jnp.clip(
                v, 0.0, float(NUM_BUCKETS - 1)).astype(jnp.int32)
            return carry

        lax.fori_loop(0, CHUNK // 16, idx_body, 0)

    def fire_gathers(b):
        for j in range(NG):
            pltpu.async_copy(
                table_s.at[idx_v[b].at[pl.ds(j * GSIZE, GSIZE)]],
                rows_v[b].at[pl.ds(j * GSIZE, GSIZE)],
                gather_sem,
            )

    def wait_gathers(b):
        for j in range(NG):
            pltpu.make_async_copy(
                table_s.at[idx_v[b].at[pl.ds(j * GSIZE, GSIZE)]],
                rows_v[b].at[pl.ds(j * GSIZE, GSIZE)],
                gather_sem,
            ).wait()

    def drain_store(b):
        pltpu.make_async_copy(
            rows_v[b], out_hbm.at[pl.ds(base, CHUNK)], store_sem
        ).wait()

    pltpu.sync_copy(dist_hbm.at[pl.ds(base, CHUNK)], dist_v[0])
    compute_idx(0)
    fire_gathers(0)

    def outer(gg, carry):
        for b in range(2):
            g = gg * 2 + b
            nb = 1 - b
            off = base + g * CHUNK
            more = g + 1 < NCHUNK

            @pl.when(more)
            def _prefetch_dist():
                pltpu.async_copy(
                    dist_hbm.at[pl.ds(off + CHUNK, CHUNK)], dist_v[nb],
                    dist_sem)

            @pl.when(g >= 1)
            def _drain_prev_store():
                drain_store(nb)

            wait_gathers(b)
            pltpu.async_copy(rows_v[b], out_hbm.at[pl.ds(off, CHUNK)],
                             store_sem)

            @pl.when(more)
            def _next_chunk():
                pltpu.make_async_copy(
                    dist_hbm.at[pl.ds(off + CHUNK, CHUNK)], dist_v[nb],
                    dist_sem).wait()
                compute_idx(nb)
                fire_gathers(nb)

        return carry

    lax.fori_loop(0, NCHUNK // 2, outer, 0)
    drain_store((NCHUNK - 1) % 2)


def kernel(distance_matrix, table):
    dist_flat = distance_matrix.reshape(TOTAL)
    mesh = plsc.VectorSubcoreMesh(core_axis_name="c", subcore_axis_name="s")
    k = functools.partial(
        pl.kernel,
        out_type=jax.ShapeDtypeStruct((TOTAL, EMB), jnp.float32),
        mesh=mesh,
        scratch_types=[
            pltpu.VMEM((CHUNK,), jnp.float32),
            pltpu.VMEM((CHUNK,), jnp.float32),
            pltpu.VMEM((CHUNK,), jnp.int32),
            pltpu.VMEM((CHUNK,), jnp.int32),
            pltpu.VMEM((CHUNK, EMB), jnp.float32),
            pltpu.VMEM((CHUNK, EMB), jnp.float32),
            pltpu.VMEM_SHARED((NUM_BUCKETS, EMB), jnp.float32),
            pltpu.SemaphoreType.DMA,
            pltpu.SemaphoreType.DMA,
            pltpu.SemaphoreType.DMA,
        ],
        compiler_params=pltpu.CompilerParams(use_tc_tiling_on_sc=False),
    )(_body)
    out = k(dist_flat, table)
    return out.reshape(B, N, N, EMB)
